# baseline (speedup 1.0000x reference)
import functools

import jax
import jax.numpy as jnp
from jax import lax
from jax.experimental import pallas as pl
from jax.experimental.pallas import tpu as pltpu
from jax.experimental.pallas import tpu_sc as plsc

N = 10000
D_IN = 128
D_HID = 16
E = 320000

NC, NS, L = 2, 16, 16
NW = NC * NS
CH = 128
NCH = 80
EPT = CH * NCH
E_PAD = EPT * NW
EROW = E_PAD // 128
NBUF = 4

N_PAD = 10048
NPR = N_PAD * D_HID // 128

_mesh = plsc.VectorSubcoreMesh(core_axis_name="c", subcore_axis_name="s")
_sc_params = pltpu.CompilerParams(use_tc_tiling_on_sc=False)


@functools.partial(
    pl.kernel,
    out_type=jax.ShapeDtypeStruct((NC, N_PAD, D_HID), jnp.float32),
    mesh=_mesh,
    compiler_params=_sc_params,
    scratch_types=[
        pltpu.VMEM((NCH, CH), jnp.int32),
        pltpu.VMEM((CH, D_HID), jnp.float32),
        pltpu.VMEM_SHARED((N_PAD, D_HID), jnp.float32),
        pltpu.SemaphoreType.DMA,
    ],
)
def _sc_deg(epk_hbm, zero_hbm, ones_hbm, out_hbm, idx_v, ones_v, acc_sh, sem):
    cid = lax.axis_index("c")
    sid = lax.axis_index("s")
    wid = sid * NC + cid
    pltpu.sync_copy(epk_hbm.at[1, wid], idx_v)
    pltpu.sync_copy(ones_hbm, ones_v)

    @pl.when(sid == 0)
    def _():
        pltpu.sync_copy(zero_hbm, acc_sh)

    plsc.subcore_barrier()

    def fire(c, carry):
        pltpu.async_copy(ones_v, acc_sh.at[idx_v.at[c]], sem, add=True)
        return carry

    lax.fori_loop(0, NCH, fire, 0)

    def drain(c, carry):
        pltpu.make_async_copy(ones_v, acc_sh.at[idx_v.at[0]], sem).wait()
        return carry

    lax.fori_loop(0, NCH, drain, 0)
    plsc.subcore_barrier()

    @pl.when(sid == 0)
    def _():
        pltpu.sync_copy(acc_sh, out_hbm.at[cid])


@functools.partial(
    pl.kernel,
    out_type=jax.ShapeDtypeStruct((NC, N_PAD, D_HID), jnp.float32),
    mesh=_mesh,
    compiler_params=_sc_params,
    scratch_types=[
        pltpu.VMEM((NCH, CH), jnp.int32),
        pltpu.VMEM((NCH, CH), jnp.int32),
        [pltpu.VMEM((CH, D_HID), jnp.float32)] * NBUF,
        pltpu.VMEM_SHARED((N_PAD, D_HID), jnp.float32),
        pltpu.VMEM_SHARED((N_PAD, D_HID), jnp.float32),
        [pltpu.SemaphoreType.DMA] * NBUF,
        [pltpu.SemaphoreType.DMA] * NBUF,
    ],
)
def _sc_scatter(hs_hbm, epk_hbm, zero_hbm, out_hbm,
                isrc_v, idst_v, rows, hs_sh, acc_sh, sg, ss):
    cid = lax.axis_index("c")
    sid = lax.axis_index("s")
    wid = sid * NC + cid
    pltpu.sync_copy(epk_hbm.at[0, wid], isrc_v)
    pltpu.sync_copy(epk_hbm.at[1, wid], idst_v)

    @pl.when(sid == 0)
    def _():
        pltpu.sync_copy(zero_hbm, acc_sh)

    @pl.when(sid == 1)
    def _():
        pltpu.sync_copy(hs_hbm, hs_sh)

    plsc.subcore_barrier()

    def gather(c, slot):
        tab = hs_hbm if slot == 0 else hs_sh
        pltpu.async_copy(tab.at[isrc_v.at[c]], rows[slot], sg[slot])

    def gather_wait(slot):
        tab = hs_hbm if slot == 0 else hs_sh
        pltpu.make_async_copy(tab.at[isrc_v.at[0]], rows[slot],
                              sg[slot]).wait()

    def scat(c, slot):
        pltpu.async_copy(rows[slot], acc_sh.at[idst_v.at[c]], ss[slot],
                         add=True)

    def scat_wait(slot):
        pltpu.make_async_copy(rows[slot], acc_sh.at[idst_v.at[0]],
                              ss[slot]).wait()

    gather(0, 0)
    for c in range(NBUF - 1):
        gather(c + 1, c + 1)
        gather_wait(c)
        scat(c, c)

    for c in range(NBUF - 1, (NBUF - 1) * NBUF):
        b = c % NBUF
        nb = (c + 1) % NBUF
        scat_wait(nb)
        gather(c + 1, nb)
        gather_wait(b)
        scat(c, b)

    def outer(g, carry):
        for b in range(NBUF):
            c = g * NBUF + b
            nb = (b + 1) % NBUF
            scat_wait(nb)

            @pl.when(c + 1 < NCH)
            def _():
                gather(c + 1, nb)

            gather_wait(b)
            scat(c, b)
        return carry

    lax.fori_loop(NBUF - 1, NCH // NBUF, outer, 0)

    for k in range(NBUF - 1):
        scat_wait((NCH - NBUF + 1 + k) % NBUF)
    plsc.subcore_barrier()

    @pl.when(sid == 0)
    def _():
        pltpu.sync_copy(acc_sh, out_hbm.at[cid])


def _tc_repack_body(e_ref, o_ref):
    e = e_ref[...]
    j = lax.broadcasted_iota(jnp.int32, (1, E_PAD - E), 1)
    pad = jnp.concatenate([
        jnp.zeros((1, E_PAD - E), jnp.int32),
        N + j % (N_PAD - N)], axis=0)
    o_ref[...] = jnp.concatenate([e, pad], axis=1).reshape(2, EROW, 128)


def _tc_prescale_body(xw_ref, wb_ref, deg_ref, hs_ref, dinv_ref):
    h = jnp.dot(xw_ref[...], wb_ref[...], preferred_element_type=jnp.float32)
    dinv = lax.rsqrt(deg_ref[0] + deg_ref[1] + 1.0)
    dinv_ref[...] = dinv
    hs_ref[...] = h * dinv


def _tc_final_body(p_ref, hs_ref, dinv_ref, b_ref, o_ref):
    s = p_ref[0] + p_ref[1] + hs_ref[...]
    o_ref[...] = jnp.maximum(s * dinv_ref[...] + b_ref[...], 0.0)


def kernel(x, edge_index, W, b):
    epk2 = pl.pallas_call(
        _tc_repack_body,
        out_shape=jax.ShapeDtypeStruct((2, EROW, 128), jnp.int32),
    )(edge_index)
    epk = epk2.reshape(2, NW, NCH, CH)
    zeros = jnp.zeros((N_PAD, D_HID), jnp.float32)
    ones = jnp.ones((CH, D_HID), jnp.float32)
    xw = jnp.concatenate(
        [x, jnp.zeros((N_PAD - N, D_IN), x.dtype)]).reshape(NPR, 8 * D_IN)
    wb = jnp.kron(jnp.eye(8, dtype=W.dtype), W)
    b128 = jnp.tile(b, 8).reshape(1, 128)

    deg_part = _sc_deg(epk, zeros, ones)
    deg2 = deg_part.reshape(NC, NPR, 128)

    hs2, dinv2 = pl.pallas_call(
        _tc_prescale_body,
        out_shape=[jax.ShapeDtypeStruct((NPR, 128), jnp.float32),
                   jax.ShapeDtypeStruct((NPR, 128), jnp.float32)],
    )(xw, wb, deg2)

    part = _sc_scatter(hs2.reshape(N_PAD, D_HID), epk, zeros)
    part2 = part.reshape(NC, NPR, 128)

    o2 = pl.pallas_call(
        _tc_final_body,
        out_shape=jax.ShapeDtypeStruct((NPR, 128), jnp.float32),
    )(part2, hs2, dinv2, b128)
    return o2.reshape(N_PAD, D_HID)[:N]


# speedup vs baseline: 1.1296x; 1.1296x over previous
import functools

import jax
import jax.numpy as jnp
from jax import lax
from jax.experimental import pallas as pl
from jax.experimental.pallas import tpu as pltpu
from jax.experimental.pallas import tpu_sc as plsc

N = 10000
D_IN = 128
D_HID = 16
E = 320000

NC, NS, L = 2, 16, 16
NW = NC * NS
CH = 128
NCH = 80
EPT = CH * NCH
E_PAD = EPT * NW
EROW = E_PAD // 128
NBUF = 4

N_PAD = 10048
NPR = N_PAD * D_HID // 128

_mesh = plsc.VectorSubcoreMesh(core_axis_name="c", subcore_axis_name="s")
_sc_params = pltpu.CompilerParams(use_tc_tiling_on_sc=False)


@functools.partial(
    pl.kernel,
    out_type=jax.ShapeDtypeStruct((NC, N_PAD, D_HID), jnp.float32),
    mesh=_mesh,
    compiler_params=_sc_params,
    scratch_types=[
        pltpu.VMEM((NCH, CH), jnp.int32),
        pltpu.VMEM((CH, D_HID), jnp.float32),
        pltpu.VMEM_SHARED((N_PAD, D_HID), jnp.float32),
        pltpu.SemaphoreType.DMA,
    ],
)
def _sc_deg(epk_hbm, zero_hbm, ones_hbm, out_hbm, idx_v, ones_v, acc_sh, sem):
    cid = lax.axis_index("c")
    sid = lax.axis_index("s")
    wid = sid * NC + cid
    pltpu.sync_copy(epk_hbm.at[1, wid], idx_v)
    pltpu.sync_copy(ones_hbm, ones_v)

    @pl.when(sid == 0)
    def _():
        pltpu.sync_copy(zero_hbm, acc_sh)

    plsc.subcore_barrier()

    def fire(c, carry):
        pltpu.async_copy(ones_v, acc_sh.at[idx_v.at[c]], sem, add=True)
        return carry

    lax.fori_loop(0, NCH, fire, 0)

    def drain(c, carry):
        pltpu.make_async_copy(ones_v, acc_sh.at[idx_v.at[0]], sem).wait()
        return carry

    lax.fori_loop(0, NCH, drain, 0)
    plsc.subcore_barrier()

    @pl.when(sid == 0)
    def _():
        pltpu.sync_copy(acc_sh, out_hbm.at[cid])


@functools.partial(
    pl.kernel,
    out_type=jax.ShapeDtypeStruct((NC, N_PAD, D_HID), jnp.float32),
    mesh=_mesh,
    compiler_params=_sc_params,
    scratch_types=[
        pltpu.VMEM((NCH, CH), jnp.int32),
        pltpu.VMEM((NCH, CH), jnp.int32),
        [pltpu.VMEM((CH, D_HID), jnp.float32)] * NBUF,
        pltpu.VMEM_SHARED((N_PAD, D_HID), jnp.float32),
        pltpu.VMEM_SHARED((N_PAD, D_HID), jnp.float32),
        [pltpu.SemaphoreType.DMA] * NBUF,
        [pltpu.SemaphoreType.DMA] * NBUF,
    ],
)
def _sc_scatter(hs_hbm, epk_hbm, zero_hbm, out_hbm,
                isrc_v, idst_v, rows, hs_sh, acc_sh, sg, ss):
    cid = lax.axis_index("c")
    sid = lax.axis_index("s")
    wid = sid * NC + cid
    pltpu.sync_copy(epk_hbm.at[0, wid], isrc_v)
    pltpu.sync_copy(epk_hbm.at[1, wid], idst_v)

    @pl.when(sid == 0)
    def _():
        pltpu.sync_copy(zero_hbm, acc_sh)

    @pl.when(sid == 1)
    def _():
        pltpu.sync_copy(hs_hbm, hs_sh)

    plsc.subcore_barrier()

    def gather(c, slot):
        pltpu.async_copy(hs_sh.at[isrc_v.at[c]], rows[slot], sg[slot])

    def gather_wait(slot):
        pltpu.make_async_copy(hs_sh.at[isrc_v.at[0]], rows[slot],
                              sg[slot]).wait()

    def scat(c, slot):
        pltpu.async_copy(rows[slot], acc_sh.at[idst_v.at[c]], ss[slot],
                         add=True)

    def scat_wait(slot):
        pltpu.make_async_copy(rows[slot], acc_sh.at[idst_v.at[0]],
                              ss[slot]).wait()

    gather(0, 0)
    for c in range(NBUF - 1):
        gather(c + 1, c + 1)
        gather_wait(c)
        scat(c, c)

    for c in range(NBUF - 1, (NBUF - 1) * NBUF):
        b = c % NBUF
        nb = (c + 1) % NBUF
        scat_wait(nb)
        gather(c + 1, nb)
        gather_wait(b)
        scat(c, b)

    def outer(g, carry):
        for b in range(NBUF):
            c = g * NBUF + b
            nb = (b + 1) % NBUF
            scat_wait(nb)

            @pl.when(c + 1 < NCH)
            def _():
                gather(c + 1, nb)

            gather_wait(b)
            scat(c, b)
        return carry

    lax.fori_loop(NBUF - 1, NCH // NBUF, outer, 0)

    for k in range(NBUF - 1):
        scat_wait((NCH - NBUF + 1 + k) % NBUF)
    plsc.subcore_barrier()

    @pl.when(sid == 0)
    def _():
        pltpu.sync_copy(acc_sh, out_hbm.at[cid])


def _tc_repack_body(e_ref, o_ref):
    e = e_ref[...]
    j = lax.broadcasted_iota(jnp.int32, (1, E_PAD - E), 1)
    pad = jnp.concatenate([
        jnp.zeros((1, E_PAD - E), jnp.int32),
        N + j % (N_PAD - N)], axis=0)
    o_ref[...] = jnp.concatenate([e, pad], axis=1).reshape(2, EROW, 128)


def _tc_prescale_body(xw_ref, wb_ref, deg_ref, hs_ref, dinv_ref):
    h = jnp.dot(xw_ref[...], wb_ref[...], preferred_element_type=jnp.float32)
    dinv = lax.rsqrt(deg_ref[0] + deg_ref[1] + 1.0)
    dinv_ref[...] = dinv
    hs_ref[...] = h * dinv


def _tc_final_body(p_ref, hs_ref, dinv_ref, b_ref, o_ref):
    s = p_ref[0] + p_ref[1] + hs_ref[...]
    o_ref[...] = jnp.maximum(s * dinv_ref[...] + b_ref[...], 0.0)


def kernel(x, edge_index, W, b):
    epk2 = pl.pallas_call(
        _tc_repack_body,
        out_shape=jax.ShapeDtypeStruct((2, EROW, 128), jnp.int32),
    )(edge_index)
    epk = epk2.reshape(2, NW, NCH, CH)
    zeros = jnp.zeros((N_PAD, D_HID), jnp.float32)
    ones = jnp.ones((CH, D_HID), jnp.float32)
    xw = jnp.concatenate(
        [x, jnp.zeros((N_PAD - N, D_IN), x.dtype)]).reshape(NPR, 8 * D_IN)
    wb = jnp.kron(jnp.eye(8, dtype=W.dtype), W)
    b128 = jnp.tile(b, 8).reshape(1, 128)

    deg_part = _sc_deg(epk, zeros, ones)
    deg2 = deg_part.reshape(NC, NPR, 128)

    hs2, dinv2 = pl.pallas_call(
        _tc_prescale_body,
        out_shape=[jax.ShapeDtypeStruct((NPR, 128), jnp.float32),
                   jax.ShapeDtypeStruct((NPR, 128), jnp.float32)],
    )(xw, wb, deg2)

    part = _sc_scatter(hs2.reshape(N_PAD, D_HID), epk, zeros)
    part2 = part.reshape(NC, NPR, 128)

    o2 = pl.pallas_call(
        _tc_final_body,
        out_shape=jax.ShapeDtypeStruct((NPR, 128), jnp.float32),
    )(part2, hs2, dinv2, b128)
    return o2.reshape(N_PAD, D_HID)[:N]
